# Initial kernel scaffold; baseline (speedup 1.0000x reference)
import functools

import jax
import jax.numpy as jnp
from jax import lax
from jax.experimental import pallas as pl
from jax.experimental.pallas import tpu as pltpu
from jax.experimental.pallas import tpu_sc as plsc

N = 10000
E = 320000
D_IN = 128
H2 = 128
H = 64
NCLS = 3
NCLS_PAD = 8

NC = 2
NS = 16
NW = NC * NS
CH = 128
NCH = 80
EPAD = NW * NCH * CH
NPAD = 10240
STRIPE = NPAD // NS
TRASH = N
NB = 4

_mesh = plsc.VectorSubcoreMesh(core_axis_name="c", subcore_axis_name="s")


def _deg_call(dst_idx, ones8, zeros8):
    @functools.partial(
        pl.kernel,
        out_type=jax.ShapeDtypeStruct((NC, NPAD, 8), jnp.float32),
        mesh=_mesh,
        scratch_types=[
            pltpu.VMEM((NCH, CH), jnp.int32),
            pltpu.VMEM((CH, 8), jnp.float32),
            pltpu.VMEM((CH, 8), jnp.float32),
            pltpu.VMEM((STRIPE, 8), jnp.float32),
            pltpu.VMEM_SHARED((NPAD, 8), jnp.float32),
        ] + [pltpu.SemaphoreType.DMA] * NB,
    )
    def k(dst_hbm, ones_hbm, zeros_hbm, out_hbm, dst_v, ones_v, zbuf, dbuf,
          acc, *sems):
        c = lax.axis_index("c")
        s = lax.axis_index("s")
        w = c * NS + s
        pltpu.sync_copy(dst_hbm.at[pl.ds(w * NCH, NCH)], dst_v)
        pltpu.sync_copy(ones_hbm, ones_v)
        pltpu.sync_copy(zeros_hbm, zbuf)
        base = s * STRIPE
        for kk in range(STRIPE // CH):
            pltpu.sync_copy(zbuf, acc.at[pl.ds(base + kk * CH, CH)])
        plsc.subcore_barrier()

        def group(g, carry):
            for b in range(NB):
                j = g * NB + b
                pltpu.async_copy(ones_v, acc.at[dst_v.at[j]], sems[b],
                                 add=True)
            for b in range(NB):
                pltpu.make_async_copy(ones_v, acc.at[dst_v.at[b]],
                                      sems[b]).wait()
            return carry

        lax.fori_loop(0, NCH // NB, group, 0)
        plsc.subcore_barrier()
        pltpu.sync_copy(acc.at[pl.ds(base, STRIPE)], dbuf)
        pltpu.sync_copy(dbuf, out_hbm.at[c].at[pl.ds(base, STRIPE)])

    return k(dst_idx, ones8, zeros8)


def _seg_sum_call(src_idx, dst_idx, u, zerosF):
    F = u.shape[1]

    @functools.partial(
        pl.kernel,
        out_type=jax.ShapeDtypeStruct((NC, NPAD, F), jnp.float32),
        mesh=_mesh,
        scratch_types=[
            pltpu.VMEM((NCH, CH), jnp.int32),
            pltpu.VMEM((NCH, CH), jnp.int32),
            pltpu.VMEM((NB, CH, F), jnp.float32),
            pltpu.VMEM((CH, F), jnp.float32),
            pltpu.VMEM_SHARED((NPAD, F), jnp.float32),
        ] + [pltpu.SemaphoreType.DMA] * (2 * NB),
    )
    def k(src_hbm, dst_hbm, u_hbm, zeros_hbm, out_hbm, src_v, dst_v, ring,
          zbuf, acc, *sems):
        gsem = sems[:NB]
        ssem = sems[NB:]
        c = lax.axis_index("c")
        s = lax.axis_index("s")
        w = c * NS + s
        pltpu.sync_copy(src_hbm.at[pl.ds(w * NCH, NCH)], src_v)
        pltpu.sync_copy(dst_hbm.at[pl.ds(w * NCH, NCH)], dst_v)
        pltpu.sync_copy(zeros_hbm, zbuf)
        base = s * STRIPE
        for kk in range(STRIPE // CH):
            pltpu.sync_copy(zbuf, acc.at[pl.ds(base + kk * CH, CH)])
        plsc.subcore_barrier()

        for b in range(NB):
            pltpu.async_copy(u_hbm.at[src_v.at[b]], ring.at[b], gsem[b])

        ngrp = NCH // NB

        def group(g, carry):
            for b in range(NB):
                j = g * NB + b
                pltpu.make_async_copy(u_hbm.at[src_v.at[b]], ring.at[b],
                                      gsem[b]).wait()
                pltpu.async_copy(ring.at[b], acc.at[dst_v.at[j]], ssem[b],
                                 add=True)
            for b in range(NB):
                pltpu.make_async_copy(ring.at[b], acc.at[dst_v.at[b]],
                                      ssem[b]).wait()

                @pl.when(g < ngrp - 1)
                def _():
                    jn = (g + 1) * NB + b
                    pltpu.async_copy(u_hbm.at[src_v.at[jn]], ring.at[b],
                                     gsem[b])
            return carry

        lax.fori_loop(0, ngrp, group, 0)
        plsc.subcore_barrier()
        for kk in range(STRIPE // CH):
            pltpu.sync_copy(acc.at[pl.ds(base + kk * CH, CH)], zbuf)
            pltpu.sync_copy(zbuf, out_hbm.at[c].at[pl.ds(base + kk * CH, CH)])

    return k(src_idx, dst_idx, u, zerosF)


def _pair_gather_call(src_idx, dst_idx, h):
    @functools.partial(
        pl.kernel,
        out_type=jax.ShapeDtypeStruct((2, EPAD, H), jnp.float32),
        mesh=_mesh,
        scratch_types=[
            pltpu.VMEM((NCH, CH), jnp.int32),
            pltpu.VMEM((NCH, CH), jnp.int32),
            pltpu.VMEM((NB, CH, H), jnp.float32),
            pltpu.VMEM((NB, CH, H), jnp.float32),
        ] + [pltpu.SemaphoreType.DMA] * (4 * NB),
    )
    def k(src_hbm, dst_hbm, h_hbm, out_hbm, src_v, dst_v, ringa, ringb,
          *sems):
        gsa = sems[:NB]
        gsb = sems[NB:2 * NB]
        wsa = sems[2 * NB:3 * NB]
        wsb = sems[3 * NB:]
        c = lax.axis_index("c")
        s = lax.axis_index("s")
        w = c * NS + s
        pltpu.sync_copy(src_hbm.at[pl.ds(w * NCH, NCH)], src_v)
        pltpu.sync_copy(dst_hbm.at[pl.ds(w * NCH, NCH)], dst_v)
        obase = w * NCH * CH

        for b in range(NB):
            pltpu.async_copy(h_hbm.at[src_v.at[b]], ringa.at[b], gsa[b])
            pltpu.async_copy(h_hbm.at[dst_v.at[b]], ringb.at[b], gsb[b])

        ngrp = NCH // NB

        def group(g, carry):
            for b in range(NB):
                j = g * NB + b
                pltpu.make_async_copy(h_hbm.at[src_v.at[b]], ringa.at[b],
                                      gsa[b]).wait()
                pltpu.async_copy(ringa.at[b],
                                 out_hbm.at[0].at[pl.ds(obase + j * CH, CH)],
                                 wsa[b])
                pltpu.make_async_copy(h_hbm.at[dst_v.at[b]], ringb.at[b],
                                      gsb[b]).wait()
                pltpu.async_copy(ringb.at[b],
                                 out_hbm.at[1].at[pl.ds(obase + j * CH, CH)],
                                 wsb[b])
            for b in range(NB):
                pltpu.make_async_copy(ringa.at[b], out_hbm.at[0].at[
                    pl.ds(obase + b * CH, CH)], wsa[b]).wait()
                pltpu.make_async_copy(ringb.at[b], out_hbm.at[1].at[
                    pl.ds(obase + b * CH, CH)], wsb[b]).wait()

                @pl.when(g < ngrp - 1)
                def _():
                    jn = (g + 1) * NB + b
                    pltpu.async_copy(h_hbm.at[src_v.at[jn]], ringa.at[b],
                                     gsa[b])
                    pltpu.async_copy(h_hbm.at[dst_v.at[jn]], ringb.at[b],
                                     gsb[b])
            return carry

        lax.fori_loop(0, ngrp, group, 0)

    return k(src_idx, dst_idx, h)


_BM = 1000


def _t0_body(x_ref, w_ref, deg_ref, u_ref, dinv_ref):
    deg = deg_ref[0][:, 0:1] + deg_ref[1][:, 0:1] + 1.0
    dinv = lax.rsqrt(deg)
    xw = jnp.dot(x_ref[...], w_ref[...], preferred_element_type=jnp.float32)
    u_ref[...] = xw * dinv
    dinv_ref[...] = jnp.broadcast_to(dinv, dinv_ref.shape)


def _t0_call(x, W1, deg):
    grid = (N // _BM,)
    return pl.pallas_call(
        _t0_body,
        grid=grid,
        in_specs=[
            pl.BlockSpec((_BM, D_IN), lambda i: (i, 0)),
            pl.BlockSpec((D_IN, H2), lambda i: (0, 0)),
            pl.BlockSpec((NC, _BM, 8), lambda i: (0, i, 0)),
        ],
        out_specs=[
            pl.BlockSpec((_BM, H2), lambda i: (i, 0)),
            pl.BlockSpec((_BM, 8), lambda i: (i, 0)),
        ],
        out_shape=[
            jax.ShapeDtypeStruct((N, H2), jnp.float32),
            jax.ShapeDtypeStruct((N, 8), jnp.float32),
        ],
    )(x, W1, deg)


def _mid_body(acc_ref, u_ref, dinv_ref, w_ref, b_ref, out_ref):
    dinv = dinv_ref[:, 0:1]
    h = jnp.maximum((acc_ref[0] + acc_ref[1] + u_ref[...]) * dinv
                    + b_ref[...], 0.0)
    out_ref[...] = jnp.dot(h, w_ref[...],
                           preferred_element_type=jnp.float32) * dinv


def _mid_call(acc, u, dinv, W, b):
    fin = u.shape[1]
    fout = W.shape[1]
    grid = (N // _BM,)
    return pl.pallas_call(
        _mid_body,
        grid=grid,
        in_specs=[
            pl.BlockSpec((NC, _BM, fin), lambda i: (0, i, 0)),
            pl.BlockSpec((_BM, fin), lambda i: (i, 0)),
            pl.BlockSpec((_BM, 8), lambda i: (i, 0)),
            pl.BlockSpec((fin, fout), lambda i: (0, 0)),
            pl.BlockSpec((1, fin), lambda i: (0, 0)),
        ],
        out_specs=pl.BlockSpec((_BM, fout), lambda i: (i, 0)),
        out_shape=jax.ShapeDtypeStruct((N, fout), jnp.float32),
    )(acc, u, dinv, W, b)


def _t3_body(acc_ref, u_ref, dinv_ref, b_ref, out_ref):
    dinv = dinv_ref[:, 0:1]
    out_ref[...] = jnp.maximum(
        (acc_ref[0] + acc_ref[1] + u_ref[...]) * dinv + b_ref[...], 0.0)


def _t3_call(acc, u, dinv, b):
    grid = (N // _BM,)
    return pl.pallas_call(
        _t3_body,
        grid=grid,
        in_specs=[
            pl.BlockSpec((NC, _BM, H), lambda i: (0, i, 0)),
            pl.BlockSpec((_BM, H), lambda i: (i, 0)),
            pl.BlockSpec((_BM, 8), lambda i: (i, 0)),
            pl.BlockSpec((1, H), lambda i: (0, 0)),
        ],
        out_specs=pl.BlockSpec((_BM, H), lambda i: (i, 0)),
        out_shape=jax.ShapeDtypeStruct((N, H), jnp.float32),
    )(acc, u, dinv, b)


_BE = 3200


def _mlp_body(n1_ref, n2_ref, wp_ref, bp_ref, wl1_ref, bl1_ref, wl2_ref,
              bl2_ref, wl3_ref, bl3_ref, out_ref):
    n1 = n1_ref[0]
    n2 = n2_ref[0]
    e1 = jnp.maximum(
        jnp.dot(n1, wp_ref[0:H, :], preferred_element_type=jnp.float32)
        + jnp.dot(n2, wp_ref[H:, :], preferred_element_type=jnp.float32)
        + bp_ref[...], 0.0)
    e2 = jnp.maximum(
        jnp.dot(e1, wl1_ref[...], preferred_element_type=jnp.float32)
        + bl1_ref[...], 0.0)
    e3 = jnp.maximum(
        jnp.dot(e2, wl2_ref[...], preferred_element_type=jnp.float32)
        + bl2_ref[...], 0.0)
    out_ref[...] = jnp.dot(e3, wl3_ref[...],
                           preferred_element_type=jnp.float32) + bl3_ref[...]


def _mlp_call(e, Wp, bp, Wl1, bl1, Wl2, bl2, Wl3p, bl3p):
    grid = (E // _BE,)
    return pl.pallas_call(
        _mlp_body,
        grid=grid,
        in_specs=[
            pl.BlockSpec((1, _BE, H), lambda i: (0, i, 0)),
            pl.BlockSpec((1, _BE, H), lambda i: (1, i, 0)),
            pl.BlockSpec((H2, H2), lambda i: (0, 0)),
            pl.BlockSpec((1, H2), lambda i: (0, 0)),
            pl.BlockSpec((H2, H), lambda i: (0, 0)),
            pl.BlockSpec((1, H), lambda i: (0, 0)),
            pl.BlockSpec((H, H), lambda i: (0, 0)),
            pl.BlockSpec((1, H), lambda i: (0, 0)),
            pl.BlockSpec((H, NCLS_PAD), lambda i: (0, 0)),
            pl.BlockSpec((1, NCLS_PAD), lambda i: (0, 0)),
        ],
        out_specs=pl.BlockSpec((_BE, NCLS_PAD), lambda i: (i, 0)),
        out_shape=jax.ShapeDtypeStruct((E, NCLS_PAD), jnp.float32),
    )(e, e, Wp, bp, Wl1, bl1, Wl2, bl2, Wl3p, bl3p)


def kernel(x, edge_index, W1, b1, W2, b2, W3, b3, Wp, bp, Wl1, bl1, Wl2, bl2,
           Wl3, bl3):
    src = edge_index[0].astype(jnp.int32)
    dst = edge_index[1].astype(jnp.int32)
    src_p = jnp.concatenate(
        [src, jnp.zeros((EPAD - E,), jnp.int32)]).reshape(NW * NCH, CH)
    dst_p = jnp.concatenate(
        [dst, jnp.full((EPAD - E,), TRASH, jnp.int32)]).reshape(NW * NCH, CH)

    ones8 = jnp.ones((CH, 8), jnp.float32)
    zeros8 = jnp.zeros((CH, 8), jnp.float32)
    zeros128 = jnp.zeros((CH, H2), jnp.float32)
    zeros64 = jnp.zeros((CH, H), jnp.float32)

    deg = _deg_call(dst_p, ones8, zeros8)

    u1, dinv = _t0_call(x, W1, deg)
    acc1 = _seg_sum_call(src_p, dst_p, u1, zeros128)
    u2 = _mid_call(acc1, u1, dinv, W2, b1.reshape(1, H2))
    acc2 = _seg_sum_call(src_p, dst_p, u2, zeros128)
    u3 = _mid_call(acc2, u2, dinv, W3, b2.reshape(1, H2))
    acc3 = _seg_sum_call(src_p, dst_p, u3, zeros64)
    h3 = _t3_call(acc3, u3, dinv, b3.reshape(1, H))

    e = _pair_gather_call(src_p, dst_p, h3)

    Wl3p = jnp.pad(Wl3, ((0, 0), (0, NCLS_PAD - NCLS)))
    bl3p = jnp.pad(bl3, (0, NCLS_PAD - NCLS)).reshape(1, NCLS_PAD)
    out = _mlp_call(e, Wp, bp.reshape(1, H2), Wl1, bl1.reshape(1, H),
                    Wl2, bl2.reshape(1, H), Wl3p, bl3p)
    return out[:, :NCLS]


# speedup vs baseline: 7.1218x; 7.1218x over previous
import functools

import jax
import jax.numpy as jnp
from jax import lax
from jax.experimental import pallas as pl
from jax.experimental.pallas import tpu as pltpu
from jax.experimental.pallas import tpu_sc as plsc

N = 10000
E = 320000
D_IN = 128
H2 = 128
H = 64
NCLS = 3
NCLS_PAD = 8

NC = 2
NS = 16
NW = NC * NS
CH = 128
NCH = 80
EPAD = NW * NCH * CH
NPAD = 10240
STRIPE = NPAD // NS
TRASH = N
NB = 4

_mesh = plsc.VectorSubcoreMesh(core_axis_name="c", subcore_axis_name="s")


def _deg_call(dst_idx, ones8, zeros8):
    @functools.partial(
        pl.kernel,
        out_type=jax.ShapeDtypeStruct((NC, NPAD, 8), jnp.float32),
        mesh=_mesh,
        compiler_params=pltpu.CompilerParams(use_tc_tiling_on_sc=False),
        scratch_types=[
            pltpu.VMEM((NCH, CH), jnp.int32),
            pltpu.VMEM((CH, 8), jnp.float32),
            pltpu.VMEM((CH, 8), jnp.float32),
            pltpu.VMEM((STRIPE, 8), jnp.float32),
            pltpu.VMEM_SHARED((NPAD, 8), jnp.float32),
        ] + [pltpu.SemaphoreType.DMA] * NB,
    )
    def k(dst_hbm, ones_hbm, zeros_hbm, out_hbm, dst_v, ones_v, zbuf, dbuf,
          acc, *sems):
        c = lax.axis_index("c")
        s = lax.axis_index("s")
        w = c * NS + s
        pltpu.sync_copy(dst_hbm.at[pl.ds(w * NCH, NCH)], dst_v)
        pltpu.sync_copy(ones_hbm, ones_v)
        pltpu.sync_copy(zeros_hbm, zbuf)
        base = s * STRIPE
        for kk in range(STRIPE // CH):
            pltpu.sync_copy(zbuf, acc.at[pl.ds(base + kk * CH, CH)])
        plsc.subcore_barrier()

        def group(g, carry):
            for b in range(NB):
                j = g * NB + b
                pltpu.async_copy(ones_v, acc.at[dst_v.at[j]], sems[b],
                                 add=True)
            for b in range(NB):
                pltpu.make_async_copy(ones_v, acc.at[dst_v.at[b]],
                                      sems[b]).wait()
            return carry

        lax.fori_loop(0, NCH // NB, group, 0)
        plsc.subcore_barrier()
        pltpu.sync_copy(acc.at[pl.ds(base, STRIPE)], dbuf)
        pltpu.sync_copy(dbuf, out_hbm.at[c].at[pl.ds(base, STRIPE)])

    return k(dst_idx, ones8, zeros8)


NCH2 = 160


def _seg_sum_call(src_idx, dst_idx, u_split, zerosF2):
    F2 = u_split.shape[2]

    @functools.partial(
        pl.kernel,
        out_type=jax.ShapeDtypeStruct((NC, NPAD, F2), jnp.float32),
        mesh=_mesh,
        compiler_params=pltpu.CompilerParams(use_tc_tiling_on_sc=False),
        scratch_types=[
            pltpu.VMEM((NCH2, CH), jnp.int32),
            pltpu.VMEM((NCH2, CH), jnp.int32),
            pltpu.VMEM((NB, CH, F2), jnp.float32),
            pltpu.VMEM((CH, F2), jnp.float32),
            pltpu.VMEM_SHARED((NPAD, F2), jnp.float32),
        ] + [pltpu.SemaphoreType.DMA] * (2 * NB),
    )
    def k(src_hbm, dst_hbm, u_hbm, zeros_hbm, out_hbm, src_v, dst_v, ring,
          zbuf, acc, *sems):
        gsem = sems[:NB]
        ssem = sems[NB:]
        c = lax.axis_index("c")
        s = lax.axis_index("s")
        pltpu.sync_copy(src_hbm.at[pl.ds(s * NCH2, NCH2)], src_v)
        pltpu.sync_copy(dst_hbm.at[pl.ds(s * NCH2, NCH2)], dst_v)
        pltpu.sync_copy(zeros_hbm, zbuf)
        base = s * STRIPE
        for kk in range(STRIPE // CH):
            pltpu.sync_copy(zbuf, acc.at[pl.ds(base + kk * CH, CH)])
        plsc.subcore_barrier()

        for b in range(NB):
            pltpu.async_copy(u_hbm.at[c].at[src_v.at[b]], ring.at[b],
                             gsem[b])

        ngrp = NCH2 // NB

        def group(g, carry):
            for b in range(NB):
                j = g * NB + b
                pltpu.make_async_copy(u_hbm.at[c].at[src_v.at[b]],
                                      ring.at[b], gsem[b]).wait()
                pltpu.async_copy(ring.at[b], acc.at[dst_v.at[j]], ssem[b],
                                 add=True)
            for b in range(NB):
                pltpu.make_async_copy(ring.at[b], acc.at[dst_v.at[b]],
                                      ssem[b]).wait()

                @pl.when(g < ngrp - 1)
                def _():
                    jn = (g + 1) * NB + b
                    pltpu.async_copy(u_hbm.at[c].at[src_v.at[jn]],
                                     ring.at[b], gsem[b])
            return carry

        lax.fori_loop(0, ngrp, group, 0)
        plsc.subcore_barrier()
        for kk in range(STRIPE // CH):
            pltpu.sync_copy(acc.at[pl.ds(base + kk * CH, CH)], zbuf)
            pltpu.sync_copy(zbuf, out_hbm.at[c].at[pl.ds(base + kk * CH, CH)])

    return k(src_idx, dst_idx, u_split, zerosF2)


def _pair_gather_call(src_idx, dst_idx, h):
    @functools.partial(
        pl.kernel,
        out_type=jax.ShapeDtypeStruct((2, EPAD, H), jnp.float32),
        mesh=_mesh,
        compiler_params=pltpu.CompilerParams(use_tc_tiling_on_sc=False),
        scratch_types=[
            pltpu.VMEM((NCH, CH), jnp.int32),
            pltpu.VMEM((NCH, CH), jnp.int32),
            pltpu.VMEM((NB, CH, H), jnp.float32),
            pltpu.VMEM((NB, CH, H), jnp.float32),
        ] + [pltpu.SemaphoreType.DMA] * (4 * NB),
    )
    def k(src_hbm, dst_hbm, h_hbm, out_hbm, src_v, dst_v, ringa, ringb,
          *sems):
        gsa = sems[:NB]
        gsb = sems[NB:2 * NB]
        wsa = sems[2 * NB:3 * NB]
        wsb = sems[3 * NB:]
        c = lax.axis_index("c")
        s = lax.axis_index("s")
        w = c * NS + s
        pltpu.sync_copy(src_hbm.at[pl.ds(w * NCH, NCH)], src_v)
        pltpu.sync_copy(dst_hbm.at[pl.ds(w * NCH, NCH)], dst_v)
        obase = w * NCH * CH

        for b in range(NB):
            pltpu.async_copy(h_hbm.at[src_v.at[b]], ringa.at[b], gsa[b])
            pltpu.async_copy(h_hbm.at[dst_v.at[b]], ringb.at[b], gsb[b])

        ngrp = NCH // NB

        def group(g, carry):
            for b in range(NB):
                j = g * NB + b
                pltpu.make_async_copy(h_hbm.at[src_v.at[b]], ringa.at[b],
                                      gsa[b]).wait()
                pltpu.async_copy(ringa.at[b],
                                 out_hbm.at[0].at[pl.ds(obase + j * CH, CH)],
                                 wsa[b])
                pltpu.make_async_copy(h_hbm.at[dst_v.at[b]], ringb.at[b],
                                      gsb[b]).wait()
                pltpu.async_copy(ringb.at[b],
                                 out_hbm.at[1].at[pl.ds(obase + j * CH, CH)],
                                 wsb[b])
            for b in range(NB):
                pltpu.make_async_copy(ringa.at[b], out_hbm.at[0].at[
                    pl.ds(obase + b * CH, CH)], wsa[b]).wait()
                pltpu.make_async_copy(ringb.at[b], out_hbm.at[1].at[
                    pl.ds(obase + b * CH, CH)], wsb[b]).wait()

                @pl.when(g < ngrp - 1)
                def _():
                    jn = (g + 1) * NB + b
                    pltpu.async_copy(h_hbm.at[src_v.at[jn]], ringa.at[b],
                                     gsa[b])
                    pltpu.async_copy(h_hbm.at[dst_v.at[jn]], ringb.at[b],
                                     gsb[b])
            return carry

        lax.fori_loop(0, ngrp, group, 0)

    return k(src_idx, dst_idx, h)


_BM = 1000


def _t0_body(x_ref, w_ref, deg_ref, u_ref, dinv_ref):
    deg = deg_ref[0][:, 0:1] + deg_ref[1][:, 0:1] + 1.0
    dinv = lax.rsqrt(deg)
    xw = jnp.dot(x_ref[...], w_ref[...], preferred_element_type=jnp.float32)
    f2 = u_ref.shape[2]
    u_ref[0] = xw[:, :f2] * dinv
    u_ref[1] = xw[:, f2:] * dinv
    dinv_ref[...] = jnp.broadcast_to(dinv, dinv_ref.shape)


def _t0_call(x, W1, deg):
    grid = (N // _BM,)
    return pl.pallas_call(
        _t0_body,
        grid=grid,
        in_specs=[
            pl.BlockSpec((_BM, D_IN), lambda i: (i, 0)),
            pl.BlockSpec((D_IN, H2), lambda i: (0, 0)),
            pl.BlockSpec((NC, _BM, 8), lambda i: (0, i, 0)),
        ],
        out_specs=[
            pl.BlockSpec((NC, _BM, H2 // 2), lambda i: (0, i, 0)),
            pl.BlockSpec((_BM, 8), lambda i: (i, 0)),
        ],
        out_shape=[
            jax.ShapeDtypeStruct((NC, N, H2 // 2), jnp.float32),
            jax.ShapeDtypeStruct((N, 8), jnp.float32),
        ],
    )(x, W1, deg)


def _mid_body(acc_ref, u_ref, dinv_ref, w_ref, b_ref, out_ref):
    dinv = dinv_ref[:, 0:1]
    fin2 = u_ref.shape[2]
    h_lo = jnp.maximum((acc_ref[0] + u_ref[0]) * dinv + b_ref[:, :fin2], 0.0)
    h_hi = jnp.maximum((acc_ref[1] + u_ref[1]) * dinv + b_ref[:, fin2:], 0.0)
    xw = (jnp.dot(h_lo, w_ref[:fin2, :], preferred_element_type=jnp.float32)
          + jnp.dot(h_hi, w_ref[fin2:, :],
                    preferred_element_type=jnp.float32))
    f2 = out_ref.shape[2]
    out_ref[0] = xw[:, :f2] * dinv
    out_ref[1] = xw[:, f2:] * dinv


def _mid_call(acc, u, dinv, W, b):
    fin2 = u.shape[2]
    fout = W.shape[1]
    grid = (N // _BM,)
    return pl.pallas_call(
        _mid_body,
        grid=grid,
        in_specs=[
            pl.BlockSpec((NC, _BM, fin2), lambda i: (0, i, 0)),
            pl.BlockSpec((NC, _BM, fin2), lambda i: (0, i, 0)),
            pl.BlockSpec((_BM, 8), lambda i: (i, 0)),
            pl.BlockSpec((2 * fin2, fout), lambda i: (0, 0)),
            pl.BlockSpec((1, 2 * fin2), lambda i: (0, 0)),
        ],
        out_specs=pl.BlockSpec((NC, _BM, fout // 2), lambda i: (0, i, 0)),
        out_shape=jax.ShapeDtypeStruct((NC, N, fout // 2), jnp.float32),
    )(acc, u, dinv, W, b)


def _t3_body(acc_ref, u_ref, dinv_ref, b_ref, out_ref):
    dinv = dinv_ref[:, 0:1]
    f2 = u_ref.shape[2]
    h_lo = jnp.maximum((acc_ref[0] + u_ref[0]) * dinv + b_ref[:, :f2], 0.0)
    h_hi = jnp.maximum((acc_ref[1] + u_ref[1]) * dinv + b_ref[:, f2:], 0.0)
    out_ref[...] = jnp.concatenate([h_lo, h_hi], axis=1)


def _t3_call(acc, u, dinv, b):
    grid = (N // _BM,)
    return pl.pallas_call(
        _t3_body,
        grid=grid,
        in_specs=[
            pl.BlockSpec((NC, _BM, H // 2), lambda i: (0, i, 0)),
            pl.BlockSpec((NC, _BM, H // 2), lambda i: (0, i, 0)),
            pl.BlockSpec((_BM, 8), lambda i: (i, 0)),
            pl.BlockSpec((1, H), lambda i: (0, 0)),
        ],
        out_specs=pl.BlockSpec((_BM, H), lambda i: (i, 0)),
        out_shape=jax.ShapeDtypeStruct((N, H), jnp.float32),
    )(acc, u, dinv, b)


_BE = 3200


def _mlp_body(n1_ref, n2_ref, wp_ref, bp_ref, wl1_ref, bl1_ref, wl2_ref,
              bl2_ref, wl3_ref, bl3_ref, out_ref):
    n1 = n1_ref[0]
    n2 = n2_ref[0]
    e1 = jnp.maximum(
        jnp.dot(n1, wp_ref[0:H, :], preferred_element_type=jnp.float32)
        + jnp.dot(n2, wp_ref[H:, :], preferred_element_type=jnp.float32)
        + bp_ref[...], 0.0)
    e2 = jnp.maximum(
        jnp.dot(e1, wl1_ref[...], preferred_element_type=jnp.float32)
        + bl1_ref[...], 0.0)
    e3 = jnp.maximum(
        jnp.dot(e2, wl2_ref[...], preferred_element_type=jnp.float32)
        + bl2_ref[...], 0.0)
    out_ref[...] = jnp.dot(e3, wl3_ref[...],
                           preferred_element_type=jnp.float32) + bl3_ref[...]


def _mlp_call(e, Wp, bp, Wl1, bl1, Wl2, bl2, Wl3p, bl3p):
    grid = (E // _BE,)
    return pl.pallas_call(
        _mlp_body,
        grid=grid,
        in_specs=[
            pl.BlockSpec((1, _BE, H), lambda i: (0, i, 0)),
            pl.BlockSpec((1, _BE, H), lambda i: (1, i, 0)),
            pl.BlockSpec((H2, H2), lambda i: (0, 0)),
            pl.BlockSpec((1, H2), lambda i: (0, 0)),
            pl.BlockSpec((H2, H), lambda i: (0, 0)),
            pl.BlockSpec((1, H), lambda i: (0, 0)),
            pl.BlockSpec((H, H), lambda i: (0, 0)),
            pl.BlockSpec((1, H), lambda i: (0, 0)),
            pl.BlockSpec((H, NCLS_PAD), lambda i: (0, 0)),
            pl.BlockSpec((1, NCLS_PAD), lambda i: (0, 0)),
        ],
        out_specs=pl.BlockSpec((_BE, NCLS_PAD), lambda i: (i, 0)),
        out_shape=jax.ShapeDtypeStruct((E, NCLS_PAD), jnp.float32),
    )(e, e, Wp, bp, Wl1, bl1, Wl2, bl2, Wl3p, bl3p)


def kernel(x, edge_index, W1, b1, W2, b2, W3, b3, Wp, bp, Wl1, bl1, Wl2, bl2,
           Wl3, bl3):
    src = edge_index[0].astype(jnp.int32)
    dst = edge_index[1].astype(jnp.int32)
    src_p = jnp.concatenate(
        [src, jnp.zeros((EPAD - E,), jnp.int32)]).reshape(NW * NCH, CH)
    dst_p = jnp.concatenate(
        [dst, jnp.full((EPAD - E,), TRASH, jnp.int32)]).reshape(NW * NCH, CH)

    ones8 = jnp.ones((CH, 8), jnp.float32)
    zeros8 = jnp.zeros((CH, 8), jnp.float32)
    zeros64 = jnp.zeros((CH, H2 // 2), jnp.float32)
    zeros32 = jnp.zeros((CH, H // 2), jnp.float32)

    deg = _deg_call(dst_p, ones8, zeros8)

    u1, dinv = _t0_call(x, W1, deg)
    acc1 = _seg_sum_call(src_p, dst_p, u1, zeros64)
    u2 = _mid_call(acc1, u1, dinv, W2, b1.reshape(1, H2))
    acc2 = _seg_sum_call(src_p, dst_p, u2, zeros64)
    u3 = _mid_call(acc2, u2, dinv, W3, b2.reshape(1, H2))
    acc3 = _seg_sum_call(src_p, dst_p, u3, zeros32)
    h3 = _t3_call(acc3, u3, dinv, b3.reshape(1, H))

    e = _pair_gather_call(src_p, dst_p, h3)

    Wl3p = jnp.pad(Wl3, ((0, 0), (0, NCLS_PAD - NCLS)))
    bl3p = jnp.pad(bl3, (0, NCLS_PAD - NCLS)).reshape(1, NCLS_PAD)
    out = _mlp_call(e, Wp, bp.reshape(1, H2), Wl1, bl1.reshape(1, H),
                    Wl2, bl2.reshape(1, H), Wl3p, bl3p)
    return out[:, :NCLS]


# speedup vs baseline: 7.1844x; 1.0088x over previous
import functools

import jax
import jax.numpy as jnp
from jax import lax
from jax.experimental import pallas as pl
from jax.experimental.pallas import tpu as pltpu
from jax.experimental.pallas import tpu_sc as plsc

N = 10000
E = 320000
D_IN = 128
H2 = 128
H = 64
NCLS = 3
NCLS_PAD = 8

NC = 2
NS = 16
NW = NC * NS
CH = 128
NCH = 80
EPAD = NW * NCH * CH
NPAD = 10240
STRIPE = NPAD // NS
TRASH = N
NB = 4

_mesh = plsc.VectorSubcoreMesh(core_axis_name="c", subcore_axis_name="s")


def _deg_call(dst_idx, ones8, zeros8):
    @functools.partial(
        pl.kernel,
        out_type=jax.ShapeDtypeStruct((NC, NPAD, 8), jnp.float32),
        mesh=_mesh,
        compiler_params=pltpu.CompilerParams(use_tc_tiling_on_sc=False),
        scratch_types=[
            pltpu.VMEM((NCH, CH), jnp.int32),
            pltpu.VMEM((CH, 8), jnp.float32),
            pltpu.VMEM((CH, 8), jnp.float32),
            pltpu.VMEM((STRIPE, 8), jnp.float32),
            pltpu.VMEM_SHARED((NPAD, 8), jnp.float32),
        ] + [pltpu.SemaphoreType.DMA] * NB,
    )
    def k(dst_hbm, ones_hbm, zeros_hbm, out_hbm, dst_v, ones_v, zbuf, dbuf,
          acc, *sems):
        c = lax.axis_index("c")
        s = lax.axis_index("s")
        w = c * NS + s
        pltpu.sync_copy(dst_hbm.at[pl.ds(w * NCH, NCH)], dst_v)
        pltpu.sync_copy(ones_hbm, ones_v)
        pltpu.sync_copy(zeros_hbm, zbuf)
        base = s * STRIPE
        for kk in range(STRIPE // CH):
            pltpu.sync_copy(zbuf, acc.at[pl.ds(base + kk * CH, CH)])
        plsc.subcore_barrier()

        def group(g, carry):
            for b in range(NB):
                j = g * NB + b
                pltpu.async_copy(ones_v, acc.at[dst_v.at[j]], sems[b],
                                 add=True)
            for b in range(NB):
                pltpu.make_async_copy(ones_v, acc.at[dst_v.at[b]],
                                      sems[b]).wait()
            return carry

        lax.fori_loop(0, NCH // NB, group, 0)
        plsc.subcore_barrier()
        pltpu.sync_copy(acc.at[pl.ds(base, STRIPE)], dbuf)
        pltpu.sync_copy(dbuf, out_hbm.at[c].at[pl.ds(base, STRIPE)])

    return k(dst_idx, ones8, zeros8)


NCH2 = 160


def _seg_sum_call(src_idx, dst_idx, u_split, zerosF2):
    F2 = u_split.shape[2]

    @functools.partial(
        pl.kernel,
        out_type=jax.ShapeDtypeStruct((NC, NPAD, F2), jnp.float32),
        mesh=_mesh,
        compiler_params=pltpu.CompilerParams(use_tc_tiling_on_sc=False),
        scratch_types=[
            pltpu.VMEM((NCH2, CH), jnp.int32),
            pltpu.VMEM((NCH2, CH), jnp.int32),
            pltpu.VMEM((NB, CH, F2), jnp.float32),
            pltpu.VMEM((CH, F2), jnp.float32),
            pltpu.VMEM_SHARED((NPAD, F2), jnp.float32),
        ] + [pltpu.SemaphoreType.DMA] * (2 * NB),
    )
    def k(src_hbm, dst_hbm, u_hbm, zeros_hbm, out_hbm, src_v, dst_v, ring,
          zbuf, acc, *sems):
        gsem = sems[:NB]
        ssem = sems[NB:]
        c = lax.axis_index("c")
        s = lax.axis_index("s")
        pltpu.sync_copy(src_hbm.at[pl.ds(s * NCH2, NCH2)], src_v)
        pltpu.sync_copy(dst_hbm.at[pl.ds(s * NCH2, NCH2)], dst_v)
        pltpu.sync_copy(zeros_hbm, zbuf)
        base = s * STRIPE
        for kk in range(STRIPE // CH):
            pltpu.sync_copy(zbuf, acc.at[pl.ds(base + kk * CH, CH)])
        plsc.subcore_barrier()

        for b in range(NB):
            pltpu.async_copy(u_hbm.at[c].at[src_v.at[b]], ring.at[b],
                             gsem[b])

        ngrp = NCH2 // NB

        def group(g, carry):
            for b in range(NB):
                j = g * NB + b
                pltpu.make_async_copy(u_hbm.at[c].at[src_v.at[b]],
                                      ring.at[b], gsem[b]).wait()
                pltpu.async_copy(ring.at[b], acc.at[dst_v.at[j]], ssem[b],
                                 add=True)
            for b in range(NB):
                pltpu.make_async_copy(ring.at[b], acc.at[dst_v.at[b]],
                                      ssem[b]).wait()

                @pl.when(g < ngrp - 1)
                def _():
                    jn = (g + 1) * NB + b
                    pltpu.async_copy(u_hbm.at[c].at[src_v.at[jn]],
                                     ring.at[b], gsem[b])
            return carry

        lax.fori_loop(0, ngrp, group, 0)
        plsc.subcore_barrier()
        for kk in range(STRIPE // CH):
            pltpu.sync_copy(acc.at[pl.ds(base + kk * CH, CH)], zbuf)
            pltpu.sync_copy(zbuf, out_hbm.at[c].at[pl.ds(base + kk * CH, CH)])

    return k(src_idx, dst_idx, u_split, zerosF2)


def _pair_gather_call(src_idx, dst_idx, h):
    @functools.partial(
        pl.kernel,
        out_type=jax.ShapeDtypeStruct((2, EPAD, H), jnp.float32),
        mesh=_mesh,
        compiler_params=pltpu.CompilerParams(use_tc_tiling_on_sc=False),
        scratch_types=[
            pltpu.VMEM((NCH, CH), jnp.int32),
            pltpu.VMEM((NCH, CH), jnp.int32),
            pltpu.VMEM((NB, CH, H), jnp.float32),
            pltpu.VMEM((NB, CH, H), jnp.float32),
            pltpu.VMEM_SHARED((N, H), jnp.float32),
        ] + [pltpu.SemaphoreType.DMA] * (4 * NB),
    )
    def k(src_hbm, dst_hbm, h_hbm, out_hbm, src_v, dst_v, ringa, ringb,
          h_sp, *sems):
        gsa = sems[:NB]
        gsb = sems[NB:2 * NB]
        wsa = sems[2 * NB:3 * NB]
        wsb = sems[3 * NB:]
        c = lax.axis_index("c")
        s = lax.axis_index("s")
        w = c * NS + s
        pltpu.sync_copy(src_hbm.at[pl.ds(w * NCH, NCH)], src_v)
        pltpu.sync_copy(dst_hbm.at[pl.ds(w * NCH, NCH)], dst_v)
        pltpu.sync_copy(h_hbm.at[pl.ds(s * (N // NS), N // NS)],
                        h_sp.at[pl.ds(s * (N // NS), N // NS)])
        plsc.subcore_barrier()
        obase = w * NCH * CH

        for b in range(NB):
            pltpu.async_copy(h_sp.at[src_v.at[b]], ringa.at[b], gsa[b])
            pltpu.async_copy(h_sp.at[dst_v.at[b]], ringb.at[b], gsb[b])

        ngrp = NCH // NB

        def group(g, carry):
            for b in range(NB):
                j = g * NB + b
                pltpu.make_async_copy(h_hbm.at[src_v.at[b]], ringa.at[b],
                                      gsa[b]).wait()
                pltpu.async_copy(ringa.at[b],
                                 out_hbm.at[0].at[pl.ds(obase + j * CH, CH)],
                                 wsa[b])
                pltpu.make_async_copy(h_hbm.at[dst_v.at[b]], ringb.at[b],
                                      gsb[b]).wait()
                pltpu.async_copy(ringb.at[b],
                                 out_hbm.at[1].at[pl.ds(obase + j * CH, CH)],
                                 wsb[b])
            for b in range(NB):
                pltpu.make_async_copy(ringa.at[b], out_hbm.at[0].at[
                    pl.ds(obase + b * CH, CH)], wsa[b]).wait()
                pltpu.make_async_copy(ringb.at[b], out_hbm.at[1].at[
                    pl.ds(obase + b * CH, CH)], wsb[b]).wait()

                @pl.when(g < ngrp - 1)
                def _():
                    jn = (g + 1) * NB + b
                    pltpu.async_copy(h_hbm.at[src_v.at[jn]], ringa.at[b],
                                     gsa[b])
                    pltpu.async_copy(h_hbm.at[dst_v.at[jn]], ringb.at[b],
                                     gsb[b])
            return carry

        lax.fori_loop(0, ngrp, group, 0)

    return k(src_idx, dst_idx, h)


_BM = 1000


def _t0_body(x_ref, w_ref, deg_ref, u_ref, dinv_ref):
    deg = deg_ref[0][:, 0:1] + deg_ref[1][:, 0:1] + 1.0
    dinv = lax.rsqrt(deg)
    xw = jnp.dot(x_ref[...], w_ref[...], preferred_element_type=jnp.float32)
    f2 = u_ref.shape[2]
    u_ref[0] = xw[:, :f2] * dinv
    u_ref[1] = xw[:, f2:] * dinv
    dinv_ref[...] = jnp.broadcast_to(dinv, dinv_ref.shape)


def _t0_call(x, W1, deg):
    grid = (N // _BM,)
    return pl.pallas_call(
        _t0_body,
        grid=grid,
        in_specs=[
            pl.BlockSpec((_BM, D_IN), lambda i: (i, 0)),
            pl.BlockSpec((D_IN, H2), lambda i: (0, 0)),
            pl.BlockSpec((NC, _BM, 8), lambda i: (0, i, 0)),
        ],
        out_specs=[
            pl.BlockSpec((NC, _BM, H2 // 2), lambda i: (0, i, 0)),
            pl.BlockSpec((_BM, 8), lambda i: (i, 0)),
        ],
        out_shape=[
            jax.ShapeDtypeStruct((NC, N, H2 // 2), jnp.float32),
            jax.ShapeDtypeStruct((N, 8), jnp.float32),
        ],
    )(x, W1, deg)


def _mid_body(acc_ref, u_ref, dinv_ref, w_ref, b_ref, out_ref):
    dinv = dinv_ref[:, 0:1]
    fin2 = u_ref.shape[2]
    h_lo = jnp.maximum((acc_ref[0] + u_ref[0]) * dinv + b_ref[:, :fin2], 0.0)
    h_hi = jnp.maximum((acc_ref[1] + u_ref[1]) * dinv + b_ref[:, fin2:], 0.0)
    xw = (jnp.dot(h_lo, w_ref[:fin2, :], preferred_element_type=jnp.float32)
          + jnp.dot(h_hi, w_ref[fin2:, :],
                    preferred_element_type=jnp.float32))
    f2 = out_ref.shape[2]
    out_ref[0] = xw[:, :f2] * dinv
    out_ref[1] = xw[:, f2:] * dinv


def _mid_call(acc, u, dinv, W, b):
    fin2 = u.shape[2]
    fout = W.shape[1]
    grid = (N // _BM,)
    return pl.pallas_call(
        _mid_body,
        grid=grid,
        in_specs=[
            pl.BlockSpec((NC, _BM, fin2), lambda i: (0, i, 0)),
            pl.BlockSpec((NC, _BM, fin2), lambda i: (0, i, 0)),
            pl.BlockSpec((_BM, 8), lambda i: (i, 0)),
            pl.BlockSpec((2 * fin2, fout), lambda i: (0, 0)),
            pl.BlockSpec((1, 2 * fin2), lambda i: (0, 0)),
        ],
        out_specs=pl.BlockSpec((NC, _BM, fout // 2), lambda i: (0, i, 0)),
        out_shape=jax.ShapeDtypeStruct((NC, N, fout // 2), jnp.float32),
    )(acc, u, dinv, W, b)


def _t3_body(acc_ref, u_ref, dinv_ref, b_ref, out_ref):
    dinv = dinv_ref[:, 0:1]
    f2 = u_ref.shape[2]
    h_lo = jnp.maximum((acc_ref[0] + u_ref[0]) * dinv + b_ref[:, :f2], 0.0)
    h_hi = jnp.maximum((acc_ref[1] + u_ref[1]) * dinv + b_ref[:, f2:], 0.0)
    out_ref[...] = jnp.concatenate([h_lo, h_hi], axis=1)


def _t3_call(acc, u, dinv, b):
    grid = (N // _BM,)
    return pl.pallas_call(
        _t3_body,
        grid=grid,
        in_specs=[
            pl.BlockSpec((NC, _BM, H // 2), lambda i: (0, i, 0)),
            pl.BlockSpec((NC, _BM, H // 2), lambda i: (0, i, 0)),
            pl.BlockSpec((_BM, 8), lambda i: (i, 0)),
            pl.BlockSpec((1, H), lambda i: (0, 0)),
        ],
        out_specs=pl.BlockSpec((_BM, H), lambda i: (i, 0)),
        out_shape=jax.ShapeDtypeStruct((N, H), jnp.float32),
    )(acc, u, dinv, b)


_BE = 3200


def _mlp_body(n1_ref, n2_ref, wp_ref, bp_ref, wl1_ref, bl1_ref, wl2_ref,
              bl2_ref, wl3_ref, bl3_ref, out_ref):
    n1 = n1_ref[0]
    n2 = n2_ref[0]
    e1 = jnp.maximum(
        jnp.dot(n1, wp_ref[0:H, :], preferred_element_type=jnp.float32)
        + jnp.dot(n2, wp_ref[H:, :], preferred_element_type=jnp.float32)
        + bp_ref[...], 0.0)
    e2 = jnp.maximum(
        jnp.dot(e1, wl1_ref[...], preferred_element_type=jnp.float32)
        + bl1_ref[...], 0.0)
    e3 = jnp.maximum(
        jnp.dot(e2, wl2_ref[...], preferred_element_type=jnp.float32)
        + bl2_ref[...], 0.0)
    out_ref[...] = jnp.dot(e3, wl3_ref[...],
                           preferred_element_type=jnp.float32) + bl3_ref[...]


def _mlp_call(e, Wp, bp, Wl1, bl1, Wl2, bl2, Wl3p, bl3p):
    grid = (E // _BE,)
    return pl.pallas_call(
        _mlp_body,
        grid=grid,
        in_specs=[
            pl.BlockSpec((1, _BE, H), lambda i: (0, i, 0)),
            pl.BlockSpec((1, _BE, H), lambda i: (1, i, 0)),
            pl.BlockSpec((H2, H2), lambda i: (0, 0)),
            pl.BlockSpec((1, H2), lambda i: (0, 0)),
            pl.BlockSpec((H2, H), lambda i: (0, 0)),
            pl.BlockSpec((1, H), lambda i: (0, 0)),
            pl.BlockSpec((H, H), lambda i: (0, 0)),
            pl.BlockSpec((1, H), lambda i: (0, 0)),
            pl.BlockSpec((H, NCLS_PAD), lambda i: (0, 0)),
            pl.BlockSpec((1, NCLS_PAD), lambda i: (0, 0)),
        ],
        out_specs=pl.BlockSpec((_BE, NCLS_PAD), lambda i: (i, 0)),
        out_shape=jax.ShapeDtypeStruct((E, NCLS_PAD), jnp.float32),
    )(e, e, Wp, bp, Wl1, bl1, Wl2, bl2, Wl3p, bl3p)


def kernel(x, edge_index, W1, b1, W2, b2, W3, b3, Wp, bp, Wl1, bl1, Wl2, bl2,
           Wl3, bl3):
    src = edge_index[0].astype(jnp.int32)
    dst = edge_index[1].astype(jnp.int32)
    src_p = jnp.concatenate(
        [src, jnp.zeros((EPAD - E,), jnp.int32)]).reshape(NW * NCH, CH)
    dst_p = jnp.concatenate(
        [dst, jnp.full((EPAD - E,), TRASH, jnp.int32)]).reshape(NW * NCH, CH)

    ones8 = jnp.ones((CH, 8), jnp.float32)
    zeros8 = jnp.zeros((CH, 8), jnp.float32)
    zeros64 = jnp.zeros((CH, H2 // 2), jnp.float32)
    zeros32 = jnp.zeros((CH, H // 2), jnp.float32)

    deg = _deg_call(dst_p, ones8, zeros8)

    u1, dinv = _t0_call(x, W1, deg)
    acc1 = _seg_sum_call(src_p, dst_p, u1, zeros64)
    u2 = _mid_call(acc1, u1, dinv, W2, b1.reshape(1, H2))
    acc2 = _seg_sum_call(src_p, dst_p, u2, zeros64)
    u3 = _mid_call(acc2, u2, dinv, W3, b2.reshape(1, H2))
    acc3 = _seg_sum_call(src_p, dst_p, u3, zeros32)
    h3 = _t3_call(acc3, u3, dinv, b3.reshape(1, H))

    e = _pair_gather_call(src_p, dst_p, h3)

    Wl3p = jnp.pad(Wl3, ((0, 0), (0, NCLS_PAD - NCLS)))
    bl3p = jnp.pad(bl3, (0, NCLS_PAD - NCLS)).reshape(1, NCLS_PAD)
    out = _mlp_call(e, Wp, bp.reshape(1, H2), Wl1, bl1.reshape(1, H),
                    Wl2, bl2.reshape(1, H), Wl3p, bl3p)
    return out[:, :NCLS]


# speedup vs baseline: 9.9776x; 1.3888x over previous
import functools

import jax
import jax.numpy as jnp
from jax import lax
from jax.experimental import pallas as pl
from jax.experimental.pallas import tpu as pltpu
from jax.experimental.pallas import tpu_sc as plsc

N = 10000
E = 320000
D_IN = 128
H2 = 128
H = 64
NCLS = 3
NCLS_PAD = 8

NC = 2
NS = 16
NW = NC * NS
CH = 128
NCH = 80
EPAD = NW * NCH * CH
NPAD = 10240
STRIPE = NPAD // NS
TRASH = N
NB = 4

_mesh = plsc.VectorSubcoreMesh(core_axis_name="c", subcore_axis_name="s")


def _deg_call(dst_idx, ones8, zeros8):
    @functools.partial(
        pl.kernel,
        out_type=jax.ShapeDtypeStruct((NC, NPAD, 8), jnp.float32),
        mesh=_mesh,
        compiler_params=pltpu.CompilerParams(use_tc_tiling_on_sc=False),
        scratch_types=[
            pltpu.VMEM((NCH, CH), jnp.int32),
            pltpu.VMEM((CH, 8), jnp.float32),
            pltpu.VMEM((CH, 8), jnp.float32),
            pltpu.VMEM((STRIPE, 8), jnp.float32),
            pltpu.VMEM_SHARED((NPAD, 8), jnp.float32),
        ] + [pltpu.SemaphoreType.DMA] * NB,
    )
    def k(dst_hbm, ones_hbm, zeros_hbm, out_hbm, dst_v, ones_v, zbuf, dbuf,
          acc, *sems):
        c = lax.axis_index("c")
        s = lax.axis_index("s")
        w = c * NS + s
        pltpu.sync_copy(dst_hbm.at[pl.ds(w * NCH, NCH)], dst_v)
        pltpu.sync_copy(ones_hbm, ones_v)
        pltpu.sync_copy(zeros_hbm, zbuf)
        base = s * STRIPE
        for kk in range(STRIPE // CH):
            pltpu.sync_copy(zbuf, acc.at[pl.ds(base + kk * CH, CH)])
        plsc.subcore_barrier()

        def group(g, carry):
            for b in range(NB):
                j = g * NB + b
                pltpu.async_copy(ones_v, acc.at[dst_v.at[j]], sems[b],
                                 add=True)
            for b in range(NB):
                pltpu.make_async_copy(ones_v, acc.at[dst_v.at[b]],
                                      sems[b]).wait()
            return carry

        lax.fori_loop(0, NCH // NB, group, 0)
        plsc.subcore_barrier()
        pltpu.sync_copy(acc.at[pl.ds(base, STRIPE)], dbuf)
        pltpu.sync_copy(dbuf, out_hbm.at[c].at[pl.ds(base, STRIPE)])

    return k(dst_idx, ones8, zeros8)


NCH2 = 160


def _seg_sum_call(src_idx, dst_idx, u_split, zerosF2):
    F2 = u_split.shape[2]

    @functools.partial(
        pl.kernel,
        out_type=jax.ShapeDtypeStruct((NC, NPAD, F2), jnp.float32),
        mesh=_mesh,
        compiler_params=pltpu.CompilerParams(use_tc_tiling_on_sc=False),
        scratch_types=[
            pltpu.VMEM((NCH2, CH), jnp.int32),
            pltpu.VMEM((NCH2, CH), jnp.int32),
            pltpu.VMEM((NB, CH, F2), jnp.float32),
            pltpu.VMEM((CH, F2), jnp.float32),
            pltpu.VMEM_SHARED((NPAD, F2), jnp.float32),
        ] + [pltpu.SemaphoreType.DMA] * (2 * NB),
    )
    def k(src_hbm, dst_hbm, u_hbm, zeros_hbm, out_hbm, src_v, dst_v, ring,
          zbuf, acc, *sems):
        gsem = sems[:NB]
        ssem = sems[NB:]
        c = lax.axis_index("c")
        s = lax.axis_index("s")
        pltpu.sync_copy(src_hbm.at[pl.ds(s * NCH2, NCH2)], src_v)
        pltpu.sync_copy(dst_hbm.at[pl.ds(s * NCH2, NCH2)], dst_v)
        pltpu.sync_copy(zeros_hbm, zbuf)
        base = s * STRIPE
        for kk in range(STRIPE // CH):
            pltpu.sync_copy(zbuf, acc.at[pl.ds(base + kk * CH, CH)])
        plsc.subcore_barrier()

        for b in range(NB):
            pltpu.async_copy(u_hbm.at[c].at[src_v.at[b]], ring.at[b],
                             gsem[b])

        ngrp = NCH2 // NB

        def group(g, carry):
            for b in range(NB):
                j = g * NB + b
                pltpu.make_async_copy(u_hbm.at[c].at[src_v.at[b]],
                                      ring.at[b], gsem[b]).wait()
                pltpu.async_copy(ring.at[b], acc.at[dst_v.at[j]], ssem[b],
                                 add=True)
            for b in range(NB):
                pltpu.make_async_copy(ring.at[b], acc.at[dst_v.at[b]],
                                      ssem[b]).wait()

                @pl.when(g < ngrp - 1)
                def _():
                    jn = (g + 1) * NB + b
                    pltpu.async_copy(u_hbm.at[c].at[src_v.at[jn]],
                                     ring.at[b], gsem[b])
            return carry

        lax.fori_loop(0, ngrp, group, 0)
        plsc.subcore_barrier()
        for kk in range(STRIPE // CH):
            pltpu.sync_copy(acc.at[pl.ds(base + kk * CH, CH)], zbuf)
            pltpu.sync_copy(zbuf, out_hbm.at[c].at[pl.ds(base + kk * CH, CH)])

    return k(src_idx, dst_idx, u_split, zerosF2)


def _pair_gather_call(src_idx, dst_idx, h):
    @functools.partial(
        pl.kernel,
        out_type=jax.ShapeDtypeStruct((EPAD, H2), jnp.float32),
        mesh=_mesh,
        compiler_params=pltpu.CompilerParams(use_tc_tiling_on_sc=False),
        scratch_types=[
            pltpu.VMEM((NCH, CH), jnp.int32),
            pltpu.VMEM((NCH, CH), jnp.int32),
            pltpu.VMEM((NB, CH, H), jnp.float32),
            pltpu.VMEM((NB, CH, H), jnp.float32),
            pltpu.VMEM_SHARED((N, H), jnp.float32),
        ] + [pltpu.SemaphoreType.DMA] * (4 * NB),
    )
    def k(src_hbm, dst_hbm, h_hbm, out_hbm, src_v, dst_v, ringa, ringb,
          h_sp, *sems):
        gsa = sems[:NB]
        gsb = sems[NB:2 * NB]
        wsa = sems[2 * NB:3 * NB]
        wsb = sems[3 * NB:]
        c = lax.axis_index("c")
        s = lax.axis_index("s")
        w = c * NS + s
        pltpu.sync_copy(src_hbm.at[pl.ds(w * NCH, NCH)], src_v)
        pltpu.sync_copy(dst_hbm.at[pl.ds(w * NCH, NCH)], dst_v)
        pltpu.sync_copy(h_hbm.at[pl.ds(s * (N // NS), N // NS)],
                        h_sp.at[pl.ds(s * (N // NS), N // NS)])
        plsc.subcore_barrier()
        obase = w * NCH * CH

        for b in range(NB):
            pltpu.async_copy(h_sp.at[src_v.at[b]], ringa.at[b], gsa[b])
            pltpu.async_copy(h_sp.at[dst_v.at[b]], ringb.at[b], gsb[b])

        ngrp = NCH // NB

        def group(g, carry):
            for b in range(NB):
                j = g * NB + b
                rows = pl.ds(obase + j * CH, CH)
                pltpu.make_async_copy(h_sp.at[src_v.at[b]], ringa.at[b],
                                      gsa[b]).wait()
                pltpu.async_copy(ringa.at[b],
                                 out_hbm.at[rows].at[:, pl.ds(0, H)],
                                 wsa[b])
                pltpu.make_async_copy(h_sp.at[dst_v.at[b]], ringb.at[b],
                                      gsb[b]).wait()
                pltpu.async_copy(ringb.at[b],
                                 out_hbm.at[rows].at[:, pl.ds(H, H)],
                                 wsb[b])
            for b in range(NB):
                rows = pl.ds(obase + b * CH, CH)
                pltpu.make_async_copy(ringa.at[b], out_hbm.at[rows].at[
                    :, pl.ds(0, H)], wsa[b]).wait()
                pltpu.make_async_copy(ringb.at[b], out_hbm.at[rows].at[
                    :, pl.ds(H, H)], wsb[b]).wait()

                @pl.when(g < ngrp - 1)
                def _():
                    jn = (g + 1) * NB + b
                    pltpu.async_copy(h_sp.at[src_v.at[jn]], ringa.at[b],
                                     gsa[b])
                    pltpu.async_copy(h_sp.at[dst_v.at[jn]], ringb.at[b],
                                     gsb[b])
            return carry

        lax.fori_loop(0, ngrp, group, 0)

    return k(src_idx, dst_idx, h)


_BM = 1000


def _t0_body(x_ref, w_ref, deg_ref, u_ref, dinv_ref):
    deg = deg_ref[0][:, 0:1] + deg_ref[1][:, 0:1] + 1.0
    dinv = lax.rsqrt(deg)
    xw = jnp.dot(x_ref[...], w_ref[...], preferred_element_type=jnp.float32)
    f2 = u_ref.shape[2]
    u_ref[0] = xw[:, :f2] * dinv
    u_ref[1] = xw[:, f2:] * dinv
    dinv_ref[...] = jnp.broadcast_to(dinv, dinv_ref.shape)


def _t0_call(x, W1, deg):
    grid = (N // _BM,)
    return pl.pallas_call(
        _t0_body,
        grid=grid,
        in_specs=[
            pl.BlockSpec((_BM, D_IN), lambda i: (i, 0)),
            pl.BlockSpec((D_IN, H2), lambda i: (0, 0)),
            pl.BlockSpec((NC, _BM, 8), lambda i: (0, i, 0)),
        ],
        out_specs=[
            pl.BlockSpec((NC, _BM, H2 // 2), lambda i: (0, i, 0)),
            pl.BlockSpec((_BM, 8), lambda i: (i, 0)),
        ],
        out_shape=[
            jax.ShapeDtypeStruct((NC, N, H2 // 2), jnp.float32),
            jax.ShapeDtypeStruct((N, 8), jnp.float32),
        ],
    )(x, W1, deg)


def _mid_body(acc_ref, u_ref, dinv_ref, w_ref, b_ref, out_ref):
    dinv = dinv_ref[:, 0:1]
    fin2 = u_ref.shape[2]
    h_lo = jnp.maximum((acc_ref[0] + u_ref[0]) * dinv + b_ref[:, :fin2], 0.0)
    h_hi = jnp.maximum((acc_ref[1] + u_ref[1]) * dinv + b_ref[:, fin2:], 0.0)
    xw = (jnp.dot(h_lo, w_ref[:fin2, :], preferred_element_type=jnp.float32)
          + jnp.dot(h_hi, w_ref[fin2:, :],
                    preferred_element_type=jnp.float32))
    f2 = out_ref.shape[2]
    out_ref[0] = xw[:, :f2] * dinv
    out_ref[1] = xw[:, f2:] * dinv


def _mid_call(acc, u, dinv, W, b):
    fin2 = u.shape[2]
    fout = W.shape[1]
    grid = (N // _BM,)
    return pl.pallas_call(
        _mid_body,
        grid=grid,
        in_specs=[
            pl.BlockSpec((NC, _BM, fin2), lambda i: (0, i, 0)),
            pl.BlockSpec((NC, _BM, fin2), lambda i: (0, i, 0)),
            pl.BlockSpec((_BM, 8), lambda i: (i, 0)),
            pl.BlockSpec((2 * fin2, fout), lambda i: (0, 0)),
            pl.BlockSpec((1, 2 * fin2), lambda i: (0, 0)),
        ],
        out_specs=pl.BlockSpec((NC, _BM, fout // 2), lambda i: (0, i, 0)),
        out_shape=jax.ShapeDtypeStruct((NC, N, fout // 2), jnp.float32),
    )(acc, u, dinv, W, b)


def _t3_body(acc_ref, u_ref, dinv_ref, b_ref, out_ref):
    dinv = dinv_ref[:, 0:1]
    f2 = u_ref.shape[2]
    h_lo = jnp.maximum((acc_ref[0] + u_ref[0]) * dinv + b_ref[:, :f2], 0.0)
    h_hi = jnp.maximum((acc_ref[1] + u_ref[1]) * dinv + b_ref[:, f2:], 0.0)
    out_ref[...] = jnp.concatenate([h_lo, h_hi], axis=1)


def _t3_call(acc, u, dinv, b):
    grid = (N // _BM,)
    return pl.pallas_call(
        _t3_body,
        grid=grid,
        in_specs=[
            pl.BlockSpec((NC, _BM, H // 2), lambda i: (0, i, 0)),
            pl.BlockSpec((NC, _BM, H // 2), lambda i: (0, i, 0)),
            pl.BlockSpec((_BM, 8), lambda i: (i, 0)),
            pl.BlockSpec((1, H), lambda i: (0, 0)),
        ],
        out_specs=pl.BlockSpec((_BM, H), lambda i: (i, 0)),
        out_shape=jax.ShapeDtypeStruct((N, H), jnp.float32),
    )(acc, u, dinv, b)


_BE = 3200


def _mlp_body(e_ref, wp_ref, bp_ref, wl1_ref, bl1_ref, wl2_ref,
              bl2_ref, wl3_ref, bl3_ref, out_ref):
    e = e_ref[...].astype(jnp.bfloat16)
    e1 = jnp.maximum(
        jnp.dot(e, wp_ref[...], preferred_element_type=jnp.float32)
        + bp_ref[...], 0.0).astype(jnp.bfloat16)
    e2 = jnp.maximum(
        jnp.dot(e1, wl1_ref[...], preferred_element_type=jnp.float32)
        + bl1_ref[...], 0.0).astype(jnp.bfloat16)
    e3 = jnp.maximum(
        jnp.dot(e2, wl2_ref[...], preferred_element_type=jnp.float32)
        + bl2_ref[...], 0.0).astype(jnp.bfloat16)
    out_ref[...] = jnp.dot(e3, wl3_ref[...],
                           preferred_element_type=jnp.float32) + bl3_ref[...]


def _mlp_call(e, Wp, bp, Wl1, bl1, Wl2, bl2, Wl3, bl3):
    grid = (E // _BE,)
    return pl.pallas_call(
        _mlp_body,
        grid=grid,
        in_specs=[
            pl.BlockSpec((_BE, H2), lambda i: (i, 0)),
            pl.BlockSpec((H2, H2), lambda i: (0, 0)),
            pl.BlockSpec((1, H2), lambda i: (0, 0)),
            pl.BlockSpec((H2, H), lambda i: (0, 0)),
            pl.BlockSpec((1, H), lambda i: (0, 0)),
            pl.BlockSpec((H, H), lambda i: (0, 0)),
            pl.BlockSpec((1, H), lambda i: (0, 0)),
            pl.BlockSpec((H, NCLS), lambda i: (0, 0)),
            pl.BlockSpec((1, NCLS), lambda i: (0, 0)),
        ],
        out_specs=pl.BlockSpec((_BE, NCLS), lambda i: (i, 0)),
        out_shape=jax.ShapeDtypeStruct((E, NCLS), jnp.float32),
    )(e, Wp.astype(jnp.bfloat16), bp.reshape(1, H2),
      Wl1.astype(jnp.bfloat16), bl1.reshape(1, H),
      Wl2.astype(jnp.bfloat16), bl2.reshape(1, H),
      Wl3.astype(jnp.bfloat16), bl3.reshape(1, NCLS))


def kernel(x, edge_index, W1, b1, W2, b2, W3, b3, Wp, bp, Wl1, bl1, Wl2, bl2,
           Wl3, bl3):
    src = edge_index[0].astype(jnp.int32)
    dst = edge_index[1].astype(jnp.int32)
    src_p = jnp.concatenate(
        [src, jnp.zeros((EPAD - E,), jnp.int32)]).reshape(NW * NCH, CH)
    dst_p = jnp.concatenate(
        [dst, jnp.full((EPAD - E,), TRASH, jnp.int32)]).reshape(NW * NCH, CH)

    ones8 = jnp.ones((CH, 8), jnp.float32)
    zeros8 = jnp.zeros((CH, 8), jnp.float32)
    zeros64 = jnp.zeros((CH, H2 // 2), jnp.float32)
    zeros32 = jnp.zeros((CH, H // 2), jnp.float32)

    deg = _deg_call(dst_p, ones8, zeros8)

    u1, dinv = _t0_call(x, W1, deg)
    acc1 = _seg_sum_call(src_p, dst_p, u1, zeros64)
    u2 = _mid_call(acc1, u1, dinv, W2, b1.reshape(1, H2))
    acc2 = _seg_sum_call(src_p, dst_p, u2, zeros64)
    u3 = _mid_call(acc2, u2, dinv, W3, b2.reshape(1, H2))
    acc3 = _seg_sum_call(src_p, dst_p, u3, zeros32)
    h3 = _t3_call(acc3, u3, dinv, b3.reshape(1, H))

    e = _pair_gather_call(src_p, dst_p, h3)

    return _mlp_call(e, Wp, bp, Wl1, bl1, Wl2, bl2, Wl3, bl3)
